# Initial kernel scaffold; baseline (speedup 1.0000x reference)
import jax
import jax.numpy as jnp
from jax.experimental import pallas as pl


def kernel(h, edge_index, edge_attr, W):
    raise NotImplementedError("write your pallas kernel here")


# speedup vs baseline: 2.6146x; 2.6146x over previous
import functools

import jax
import jax.numpy as jnp
from jax import lax
from jax.experimental import pallas as pl
from jax.experimental.pallas import tpu as pltpu
from jax.experimental.pallas import tpu_sc as plsc

N_NODES = 10000
EMBED = 256
HALF = 128
N_EDGES = 160000

NC = 2
NS = 16
LANES = 16

CHUNK = 128
E_PAD = 163840
EDGES_PER_TILE = E_PAD // NS
N_CHUNKS = EDGES_PER_TILE // CHUNK
STRIPE = 624
REM = N_NODES - STRIPE * NS
REM_BASE = STRIPE * NS


def _sc_body(row_hbm, col_hbm, attr_hbm, hs_hbm, agg_hbm,
             row_v, col_v, attr_v, gbuf, acc, sem):
    cid = lax.axis_index("c")
    sid = lax.axis_index("s")

    pltpu.sync_copy(row_hbm.at[sid], row_v)
    pltpu.sync_copy(col_hbm.at[sid], col_v)
    pltpu.sync_copy(attr_hbm.at[sid], attr_v)

    off = (cid * N_NODES).astype(jnp.int32)

    def adj_body(j, _):
        for k in range(HALF // LANES):
            sl = pl.ds(k * LANES, LANES)
            row_v[j, sl] = row_v[j, sl] + off
        return 0

    lax.fori_loop(0, N_CHUNKS, adj_body, 0)

    def zero_body(i, _):
        for k in range(HALF // LANES):
            gbuf[i, pl.ds(k * LANES, LANES)] = jnp.zeros((LANES,), jnp.float32)
        return 0

    lax.fori_loop(0, CHUNK, zero_body, 0)

    base = sid * STRIPE
    for r0 in range(0, STRIPE, CHUNK):
        n = min(CHUNK, STRIPE - r0)
        pltpu.sync_copy(gbuf.at[pl.ds(0, n)], acc.at[pl.ds(base + r0, n)])

    @pl.when(sid == 0)
    def _zero_rem():
        pltpu.sync_copy(gbuf.at[pl.ds(0, REM)], acc.at[pl.ds(REM_BASE, REM)])

    plsc.subcore_barrier()

    def chunk_body(j, _):
        pltpu.async_copy(hs_hbm.at[row_v.at[j]], gbuf, sem).wait()

        def scale_body(g, _):
            av = attr_v[j, pl.ds(g * LANES, LANES)]
            for e in range(LANES):
                a = av[e]
                r = g * LANES + e
                for k in range(HALF // LANES):
                    sl = pl.ds(k * LANES, LANES)
                    gbuf[r, sl] = gbuf[r, sl] * a
            return 0

        lax.fori_loop(0, CHUNK // LANES, scale_body, 0)

        pltpu.sync_copy(gbuf, acc.at[col_v.at[j]], add=True)
        return 0

    lax.fori_loop(0, N_CHUNKS, chunk_body, 0)

    plsc.subcore_barrier()

    pltpu.sync_copy(acc.at[pl.ds(base, STRIPE)],
                    agg_hbm.at[pl.ds(cid * N_NODES + base, STRIPE)])

    @pl.when(sid == 0)
    def _write_rem():
        pltpu.sync_copy(acc.at[pl.ds(REM_BASE, REM)],
                        agg_hbm.at[pl.ds(cid * N_NODES + REM_BASE, REM)])


@jax.jit
def _sc_aggregate(rowp, colp, attrp, hs):
    mesh = plsc.VectorSubcoreMesh(core_axis_name="c", subcore_axis_name="s")
    return pl.kernel(
        _sc_body,
        out_type=jax.ShapeDtypeStruct((NC * N_NODES, HALF), jnp.float32),
        mesh=mesh,
        scratch_types=[
            pltpu.VMEM((N_CHUNKS, CHUNK), jnp.int32),
            pltpu.VMEM((N_CHUNKS, CHUNK), jnp.int32),
            pltpu.VMEM((N_CHUNKS, CHUNK), jnp.float32),
            pltpu.VMEM((CHUNK, HALF), jnp.float32),
            pltpu.VMEM_SHARED((N_NODES, HALF), jnp.float32),
            pltpu.SemaphoreType.DMA,
        ],
    )(rowp, colp, attrp, hs)


def _mm_body_split(agg_ref, w_ref, out_ref):
    a0 = agg_ref[0]
    a1 = agg_ref[1]
    w = w_ref[...]
    p = jnp.dot(a0, w[:HALF, :], preferred_element_type=jnp.float32)
    p = p + jnp.dot(a1, w[HALF:, :], preferred_element_type=jnp.float32)
    r = jnp.maximum(p, 0.0)
    out_ref[0] = r[:, :HALF]
    out_ref[1] = r[:, HALF:]


def _mm_body_merged(agg_ref, w_ref, out_ref):
    a0 = agg_ref[0]
    a1 = agg_ref[1]
    w = w_ref[...]
    p = jnp.dot(a0, w[:HALF, :], preferred_element_type=jnp.float32)
    p = p + jnp.dot(a1, w[HALF:, :], preferred_element_type=jnp.float32)
    out_ref[...] = jnp.maximum(p, 0.0)


_MM_BLK = 1000
_MM_GRID = N_NODES // _MM_BLK


@jax.jit
def _tc_matmul_split(agg3, W):
    return pl.pallas_call(
        _mm_body_split,
        grid=(_MM_GRID,),
        in_specs=[
            pl.BlockSpec((NC, _MM_BLK, HALF), lambda i: (0, i, 0)),
            pl.BlockSpec((EMBED, EMBED), lambda i: (0, 0)),
        ],
        out_specs=pl.BlockSpec((NC, _MM_BLK, HALF), lambda i: (0, i, 0)),
        out_shape=jax.ShapeDtypeStruct((NC, N_NODES, HALF), jnp.float32),
    )(agg3, W)


@jax.jit
def _tc_matmul_merged(agg3, W):
    return pl.pallas_call(
        _mm_body_merged,
        grid=(_MM_GRID,),
        in_specs=[
            pl.BlockSpec((NC, _MM_BLK, HALF), lambda i: (0, i, 0)),
            pl.BlockSpec((EMBED, EMBED), lambda i: (0, 0)),
        ],
        out_specs=pl.BlockSpec((_MM_BLK, EMBED), lambda i: (i, 0)),
        out_shape=jax.ShapeDtypeStruct((N_NODES, EMBED), jnp.float32),
    )(agg3, W)


def kernel(h, edge_index, edge_attr, W):
    row = edge_index[0].astype(jnp.int32)
    col = edge_index[1].astype(jnp.int32)
    attr = edge_attr.astype(jnp.float32)

    pad = E_PAD - N_EDGES
    rowp = jnp.pad(row, (0, pad)).reshape(NS, N_CHUNKS, CHUNK)
    colp = jnp.pad(col, (0, pad)).reshape(NS, N_CHUNKS, CHUNK)
    attrp = jnp.pad(attr, (0, pad)).reshape(NS, N_CHUNKS, CHUNK)

    hs = jnp.concatenate([h[:, :HALF], h[:, HALF:]], axis=0)

    for layer in range(3):
        agg = _sc_aggregate(rowp, colp, attrp, hs)
        agg3 = agg.reshape(NC, N_NODES, HALF)
        if layer < 2:
            hs = _tc_matmul_split(agg3, W).reshape(NC * N_NODES, HALF)
        else:
            out = _tc_matmul_merged(agg3, W)
    return out


# speedup vs baseline: 3.2860x; 1.2568x over previous
import functools

import jax
import jax.numpy as jnp
from jax import lax
from jax.experimental import pallas as pl
from jax.experimental.pallas import tpu as pltpu
from jax.experimental.pallas import tpu_sc as plsc

N_NODES = 10000
EMBED = 256
HALF = 128
N_EDGES = 160000

NC = 2
NS = 16
LANES = 16

CHUNK = 128
E_PAD = 163840
EDGES_PER_TILE = E_PAD // NS
N_CHUNKS = EDGES_PER_TILE // CHUNK
BLKC = 16
NBLK = N_CHUNKS // BLKC
PAIRS = BLKC // 2
STRIPE = 624
REM = N_NODES - STRIPE * NS
REM_BASE = STRIPE * NS


def _sc_body(row_hbm, col_hbm, attr_hbm, hs_hbm, agg_hbm,
             row_b0, row_b1, col_b0, col_b1, attr_b0, attr_b1,
             gbuf0, gbuf1, acc, semg0, semg1, sems0, sems1, semi):
    cid = lax.axis_index("c")
    sid = lax.axis_index("s")

    row_b = (row_b0, row_b1)
    col_b = (col_b0, col_b1)
    attr_b = (attr_b0, attr_b1)
    gbufs = (gbuf0, gbuf1)
    semg = (semg0, semg1)
    sems = (sems0, sems1)

    def stage_block(b, s, sync):
        if sync:
            pltpu.sync_copy(row_hbm.at[cid, sid, b], row_b[s])
            pltpu.sync_copy(col_hbm.at[sid, b], col_b[s])
            pltpu.sync_copy(attr_hbm.at[sid, b], attr_b[s])
        else:
            pltpu.async_copy(row_hbm.at[cid, sid, b], row_b[s], semi)
            pltpu.async_copy(col_hbm.at[sid, b], col_b[s], semi)
            pltpu.async_copy(attr_hbm.at[sid, b], attr_b[s], semi)

    def drain_stage(b, s):
        pltpu.make_async_copy(row_hbm.at[cid, sid, b], row_b[s], semi).wait()
        pltpu.make_async_copy(col_hbm.at[sid, b], col_b[s], semi).wait()
        pltpu.make_async_copy(attr_hbm.at[sid, b], attr_b[s], semi).wait()

    def start_gather(s, j, g):
        pltpu.async_copy(hs_hbm.at[row_b[s].at[j]], gbufs[g], semg[g])

    def wait_gather(s, j, g):
        pltpu.make_async_copy(hs_hbm.at[row_b[s].at[j]], gbufs[g],
                              semg[g]).wait()

    def start_scatter(s, j, g):
        pltpu.async_copy(gbufs[g], acc.at[col_b[s].at[j]], sems[g], add=True)

    def wait_scatter(s, j, g):
        pltpu.make_async_copy(gbufs[g], acc.at[col_b[s].at[j]],
                              sems[g]).wait()

    def scale(s, j, g):
        buf = gbufs[g]

        def scale_body(gg, _):
            av = attr_b[s][j, pl.ds(gg * LANES, LANES)]
            for e in range(LANES):
                a = av[e]
                r = gg * LANES + e
                for k in range(HALF // LANES):
                    sl = pl.ds(k * LANES, LANES)
                    buf[r, sl] = buf[r, sl] * a
            return 0

        lax.fori_loop(0, CHUNK // LANES, scale_body, 0)

    def zero_body(i, _):
        for k in range(HALF // LANES):
            gbuf0[i, pl.ds(k * LANES, LANES)] = jnp.zeros((LANES,), jnp.float32)
        return 0

    lax.fori_loop(0, CHUNK, zero_body, 0)

    stage_block(0, 0, sync=True)

    base = sid * STRIPE
    for r0 in range(0, STRIPE, CHUNK):
        n = min(CHUNK, STRIPE - r0)
        pltpu.sync_copy(gbuf0.at[pl.ds(0, n)], acc.at[pl.ds(base + r0, n)])

    @pl.when(sid == 0)
    def _zero_rem():
        pltpu.sync_copy(gbuf0.at[pl.ds(0, REM)], acc.at[pl.ds(REM_BASE, REM)])

    plsc.subcore_barrier()

    start_gather(0, 0, 0)

    for b in range(NBLK):
        s = b % 2
        if b + 1 < NBLK:
            stage_block(b + 1, 1 - s, sync=False)

        def pair_body(i, _, b=b, s=s):
            j0 = 2 * i
            j1 = j0 + 1
            wait_gather(s, j0, 0)
            if b > 0:
                wait_scatter(s, j0, 1)
            else:
                @pl.when(i > 0)
                def _():
                    wait_scatter(s, j0, 1)
            start_gather(s, j1, 1)
            scale(s, j0, 0)
            start_scatter(s, j0, 0)

            wait_gather(s, j1, 1)
            if b + 1 < NBLK:
                @pl.when(i < PAIRS - 1)
                def _():
                    wait_scatter(s, j0, 0)
                    start_gather(s, j1 + 1, 0)

                @pl.when(i == PAIRS - 1)
                def _():
                    wait_scatter(s, j0, 0)
                    drain_stage(b + 1, 1 - s)
                    start_gather(1 - s, 0, 0)
            else:
                @pl.when(i < PAIRS - 1)
                def _():
                    wait_scatter(s, j0, 0)
                    start_gather(s, j1 + 1, 0)

            scale(s, j1, 1)
            start_scatter(s, j1, 1)
            return 0

        lax.fori_loop(0, PAIRS, pair_body, 0)

    s_last = (NBLK - 1) % 2
    wait_scatter(s_last, BLKC - 2, 0)
    wait_scatter(s_last, BLKC - 1, 1)

    plsc.subcore_barrier()

    pltpu.sync_copy(acc.at[pl.ds(base, STRIPE)],
                    agg_hbm.at[pl.ds(cid * N_NODES + base, STRIPE)])

    @pl.when(sid == 0)
    def _write_rem():
        pltpu.sync_copy(acc.at[pl.ds(REM_BASE, REM)],
                        agg_hbm.at[pl.ds(cid * N_NODES + REM_BASE, REM)])


@jax.jit
def _sc_aggregate(rowp, colp, attrp, hs):
    mesh = plsc.VectorSubcoreMesh(core_axis_name="c", subcore_axis_name="s")
    return pl.kernel(
        _sc_body,
        out_type=jax.ShapeDtypeStruct((NC * N_NODES, HALF), jnp.float32),
        mesh=mesh,
        scratch_types=[
            pltpu.VMEM((BLKC, CHUNK), jnp.int32),
            pltpu.VMEM((BLKC, CHUNK), jnp.int32),
            pltpu.VMEM((BLKC, CHUNK), jnp.int32),
            pltpu.VMEM((BLKC, CHUNK), jnp.int32),
            pltpu.VMEM((BLKC, CHUNK), jnp.float32),
            pltpu.VMEM((BLKC, CHUNK), jnp.float32),
            pltpu.VMEM((CHUNK, HALF), jnp.float32),
            pltpu.VMEM((CHUNK, HALF), jnp.float32),
            pltpu.VMEM_SHARED((N_NODES, HALF), jnp.float32),
            pltpu.SemaphoreType.DMA,
            pltpu.SemaphoreType.DMA,
            pltpu.SemaphoreType.DMA,
            pltpu.SemaphoreType.DMA,
            pltpu.SemaphoreType.DMA,
        ],
    )(rowp, colp, attrp, hs)


def _mm_body_split(agg_ref, w_ref, out_ref):
    a0 = agg_ref[0]
    a1 = agg_ref[1]
    w = w_ref[...]
    p = jnp.dot(a0, w[:HALF, :], preferred_element_type=jnp.float32)
    p = p + jnp.dot(a1, w[HALF:, :], preferred_element_type=jnp.float32)
    r = jnp.maximum(p, 0.0)
    out_ref[0] = r[:, :HALF]
    out_ref[1] = r[:, HALF:]


def _mm_body_merged(agg_ref, w_ref, out_ref):
    a0 = agg_ref[0]
    a1 = agg_ref[1]
    w = w_ref[...]
    p = jnp.dot(a0, w[:HALF, :], preferred_element_type=jnp.float32)
    p = p + jnp.dot(a1, w[HALF:, :], preferred_element_type=jnp.float32)
    out_ref[...] = jnp.maximum(p, 0.0)


_MM_BLK = 1000
_MM_GRID = N_NODES // _MM_BLK


@jax.jit
def _tc_matmul_split(agg3, W):
    return pl.pallas_call(
        _mm_body_split,
        grid=(_MM_GRID,),
        in_specs=[
            pl.BlockSpec((NC, _MM_BLK, HALF), lambda i: (0, i, 0)),
            pl.BlockSpec((EMBED, EMBED), lambda i: (0, 0)),
        ],
        out_specs=pl.BlockSpec((NC, _MM_BLK, HALF), lambda i: (0, i, 0)),
        out_shape=jax.ShapeDtypeStruct((NC, N_NODES, HALF), jnp.float32),
    )(agg3, W)


@jax.jit
def _tc_matmul_merged(agg3, W):
    return pl.pallas_call(
        _mm_body_merged,
        grid=(_MM_GRID,),
        in_specs=[
            pl.BlockSpec((NC, _MM_BLK, HALF), lambda i: (0, i, 0)),
            pl.BlockSpec((EMBED, EMBED), lambda i: (0, 0)),
        ],
        out_specs=pl.BlockSpec((_MM_BLK, EMBED), lambda i: (i, 0)),
        out_shape=jax.ShapeDtypeStruct((N_NODES, EMBED), jnp.float32),
    )(agg3, W)


def kernel(h, edge_index, edge_attr, W):
    row = edge_index[0].astype(jnp.int32)
    col = edge_index[1].astype(jnp.int32)
    attr = edge_attr.astype(jnp.float32)

    pad = E_PAD - N_EDGES
    rowp0 = jnp.pad(row, (0, pad)).reshape(NS, NBLK, BLKC, CHUNK)
    rowp = jnp.stack([rowp0, rowp0 + N_NODES])
    colp = jnp.pad(col, (0, pad)).reshape(NS, NBLK, BLKC, CHUNK)
    attrp = jnp.pad(attr, (0, pad)).reshape(NS, NBLK, BLKC, CHUNK)

    hs = jnp.concatenate([h[:, :HALF], h[:, HALF:]], axis=0)

    for layer in range(3):
        agg = _sc_aggregate(rowp, colp, attrp, hs)
        agg3 = agg.reshape(NC, N_NODES, HALF)
        if layer < 2:
            hs = _tc_matmul_split(agg3, W).reshape(NC * N_NODES, HALF)
        else:
            out = _tc_matmul_merged(agg3, W)
    return out
